# baseline (speedup 1.0000x reference)
import functools

import jax
import jax.numpy as jnp
from jax import lax
from jax.experimental import pallas as pl
from jax.experimental.pallas import tpu as pltpu
from jax.experimental.pallas import tpu_sc as plsc

B = 64
T = 128
H = 256
W = 256
C = 4
NC = 2
NS = 16
L = 16
NW = NC * NS
BPW = B // NW
NCHUNK = T // L
ROW = 128
NROWS = (B * H * W * C) // ROW


def _batch_valid(traj_v, idx_v, b):
    lane = lax.iota(jnp.int32, L)
    in_bounds = []
    cols = []
    base = b * (H * W * C)
    for i in range(NCHUNK):
        ridx = lane + (i * L)
        xi = ridx * 3
        x = plsc.load_gather(traj_v, [xi])
        y = plsc.load_gather(traj_v, [xi + 1])
        gx = (((x + 10.0) / 20.0) * float(H)).astype(jnp.int32)
        gy = (((y + 10.0) / 20.0) * float(W)).astype(jnp.int32)
        inb = (gx >= 0) & (gx < H) & (gy >= 0) & (gy < W)
        ic = jnp.clip(gx, 0, H - 1)
        jc = jnp.clip(gy, 0, W - 1)
        flat = base + ic * (W * C) + jc * C
        idx_v[pl.ds(i * L, L)] = flat >> 7
        in_bounds.append(inb)
        cols.append((jc & 31) << 2)
    return in_bounds, cols


def _reduce_batch(rows_v, in_bounds, cols):
    lane = lax.iota(jnp.int32, L)
    acc = None
    for i in range(NCHUNK):
        ridx = lane + (i * L)
        col = cols[i]
        mass = plsc.load_gather(rows_v, [ridx, col])
        for c in range(1, C):
            mass = mass + plsc.load_gather(rows_v, [ridx, col + c])
        ok = in_bounds[i] & jnp.logical_not(mass > 100.0)
        acc = ok if acc is None else (acc & ok)
    valid = jnp.all(acc).astype(jnp.int32)
    return jnp.broadcast_to(valid, (L,))


def _collision_body(traj_hbm, map_hbm, out_hbm,
                    traj_v0, traj_v1, idx_v0, idx_v1,
                    rows_v0, rows_v1, res_v, sem0, sem1):
    wid = lax.axis_index("s") * NC + lax.axis_index("c")
    b0 = wid
    b1 = wid + NW

    pltpu.sync_copy(traj_hbm.at[b0], traj_v0)
    pltpu.sync_copy(traj_hbm.at[b1], traj_v1)

    inb0, cols0 = _batch_valid(traj_v0, idx_v0, b0)
    g0 = pltpu.async_copy(map_hbm.at[idx_v0], rows_v0, sem0)
    inb1, cols1 = _batch_valid(traj_v1, idx_v1, b1)
    g1 = pltpu.async_copy(map_hbm.at[idx_v1], rows_v1, sem1)

    g0.wait()
    res_v[...] = _reduce_batch(rows_v0, inb0, cols0)
    pltpu.sync_copy(res_v, out_hbm.at[b0])
    g1.wait()
    res_v[...] = _reduce_batch(rows_v1, inb1, cols1)
    pltpu.sync_copy(res_v, out_hbm.at[b1])


@jax.jit
def _collision_sc(trajectory, map_rows):
    kfn = pl.kernel(
        _collision_body,
        out_type=jax.ShapeDtypeStruct((B, L), jnp.int32),
        mesh=plsc.VectorSubcoreMesh(
            core_axis_name="c", subcore_axis_name="s",
            num_cores=NC, num_subcores=NS),
        scratch_types=[
            pltpu.VMEM((T * 3,), jnp.float32),
            pltpu.VMEM((T * 3,), jnp.float32),
            pltpu.VMEM((T,), jnp.int32),
            pltpu.VMEM((T,), jnp.int32),
            pltpu.VMEM((T, ROW), jnp.float32),
            pltpu.VMEM((T, ROW), jnp.float32),
            pltpu.VMEM((L,), jnp.int32),
            pltpu.SemaphoreType.DMA,
            pltpu.SemaphoreType.DMA,
        ],
        compiler_params=pltpu.CompilerParams(needs_layout_passes=False),
    )
    return kfn(trajectory, map_rows)


def kernel(trajectory, affordance_map):
    traj_flat = trajectory.reshape(B, T * 3)
    map_rows = affordance_map.reshape(NROWS, ROW)
    out = _collision_sc(traj_flat, map_rows)
    return out[:, 0].astype(jnp.bool_)


# speedup vs baseline: 180.9193x; 180.9193x over previous
import jax
import jax.numpy as jnp
from jax import lax
from jax.experimental import pallas as pl
from jax.experimental.pallas import tpu as pltpu
from jax.experimental.pallas import tpu_sc as plsc

B = 64
T = 128
H = 256
W = 256
C = 4
NC = 2
NS = 16
L = 16
NW = NC * NS
ROW = 128
NROWS = (B * H * W * C) // ROW
NCHUNK = T // L


def _batch_indices(x_v, y_v, idxs, b):
    in_bounds = []
    col0 = []
    for k in range(NCHUNK):
        x = x_v[pl.ds(k * L, L)]
        y = y_v[pl.ds(k * L, L)]
        gx = (((x + 10.0) / 20.0) * float(H)).astype(jnp.int32)
        gy = (((y + 10.0) / 20.0) * float(W)).astype(jnp.int32)
        inb = (gx >= 0) & (gx < H) & (gy >= 0) & (gy < W)
        ic = jnp.clip(gx, 0, H - 1)
        jc = jnp.clip(gy, 0, W - 1)
        r0 = b * 2048 + ic * 8 + (jc >> 7) * 4
        for c in range(C):
            idxs[c][pl.ds(k * L, L)] = r0 + c
        in_bounds.append(inb)
        col0.append(jc & 127)
    return in_bounds, col0


def _reduce_batch(rows, in_bounds, col0):
    lane = lax.iota(jnp.int32, L)
    acc = None
    for k in range(NCHUNK):
        ridx = lane + k * L
        mass = plsc.load_gather(rows[0], [ridx, col0[k]])
        for c in range(1, C):
            mass = mass + plsc.load_gather(rows[c], [ridx, col0[k]])
        ok = in_bounds[k] & jnp.logical_not(mass > 100.0)
        acc = ok if acc is None else (acc & ok)
    return acc


def _collision_body(traj_hbm, map_hbm, out_hbm,
                    x_v, y_v, idx0, idx1, idx2, idx3,
                    rows0, rows1, rows2, rows3, res_v,
                    sem0, sem1, sem2, sem3):
    idxs = (idx0, idx1, idx2, idx3)
    rows = (rows0, rows1, rows2, rows3)
    sems = (sem0, sem1, sem2, sem3)
    wid = lax.axis_index("s") * NC + lax.axis_index("c")
    for b in (wid, wid + NW):
        pltpu.sync_copy(traj_hbm.at[b], x_v)
        pltpu.sync_copy(traj_hbm.at[B + b], y_v)
        inb, col0 = _batch_indices(x_v, y_v, idxs, b)
        copies = [pltpu.async_copy(map_hbm.at[idxs[c]], rows[c], sems[c])
                  for c in range(C)]
        for cp in copies:
            cp.wait()
        acc = _reduce_batch(rows, inb, col0)
        res_v[...] = jnp.broadcast_to(jnp.all(acc).astype(jnp.int32), (L,))
        pltpu.sync_copy(res_v, out_hbm.at[b])


@jax.jit
def _collision_sc(traj_planes, map_rows):
    kfn = pl.kernel(
        _collision_body,
        out_type=jax.ShapeDtypeStruct((B, L), jnp.int32),
        mesh=plsc.VectorSubcoreMesh(
            core_axis_name="c", subcore_axis_name="s",
            num_cores=NC, num_subcores=NS),
        scratch_types=[
            pltpu.VMEM((T,), jnp.float32),
            pltpu.VMEM((T,), jnp.float32),
            pltpu.VMEM((T,), jnp.int32),
            pltpu.VMEM((T,), jnp.int32),
            pltpu.VMEM((T,), jnp.int32),
            pltpu.VMEM((T,), jnp.int32),
            pltpu.VMEM((T, ROW), jnp.float32),
            pltpu.VMEM((T, ROW), jnp.float32),
            pltpu.VMEM((T, ROW), jnp.float32),
            pltpu.VMEM((T, ROW), jnp.float32),
            pltpu.VMEM((L,), jnp.int32),
            pltpu.SemaphoreType.DMA,
            pltpu.SemaphoreType.DMA,
            pltpu.SemaphoreType.DMA,
            pltpu.SemaphoreType.DMA,
        ],
        compiler_params=pltpu.CompilerParams(needs_layout_passes=False),
    )
    return kfn(traj_planes, map_rows)


def kernel(trajectory, affordance_map):
    traj_planes = trajectory.transpose(2, 0, 1).reshape(3 * B, T)
    map_rows = (affordance_map
                .reshape(B, H, 2, W // 2, C)
                .transpose(0, 1, 2, 4, 3)
                .reshape(NROWS, ROW))
    out = _collision_sc(traj_planes, map_rows)
    return out[:, 0].astype(jnp.bool_)


# speedup vs baseline: 230.7763x; 1.2756x over previous
import jax
import jax.numpy as jnp
from jax import lax
from jax.experimental import pallas as pl
from jax.experimental.pallas import tpu as pltpu
from jax.experimental.pallas import tpu_sc as plsc

B = 64
T = 128
H = 256
W = 256
C = 4
NC = 2
NS = 16
L = 16
NW = NC * NS
NCHUNK = T // L


def _batch_indices(x_v, y_v, idx_v, b):
    in_bounds = []
    for k in range(NCHUNK):
        x = x_v[pl.ds(k * L, L)]
        y = y_v[pl.ds(k * L, L)]
        gx = (((x + 10.0) / 20.0) * float(H)).astype(jnp.int32)
        gy = (((y + 10.0) / 20.0) * float(W)).astype(jnp.int32)
        inb = (gx >= 0) & (gx < H) & (gy >= 0) & (gy < W)
        ic = jnp.clip(gx, 0, H - 1)
        jc = jnp.clip(gy, 0, W - 1)
        e0 = (b * 512 + ic * 2 + (jc >> 7)) * 512 + (jc & 127)
        for c in range(C):
            idx_v[c, pl.ds(k * L, L)] = e0 + c * 128
        in_bounds.append(inb)
    return in_bounds


def _reduce_batch(dst_v, in_bounds):
    acc = None
    for k in range(NCHUNK):
        mass = dst_v[0, pl.ds(k * L, L)]
        for c in range(1, C):
            mass = mass + dst_v[c, pl.ds(k * L, L)]
        ok = in_bounds[k] & jnp.logical_not(mass > 100.0)
        acc = ok if acc is None else (acc & ok)
    return acc


def _collision_body(traj_hbm, map_hbm, out_hbm,
                    x_v, y_v, idx_a, idx_b, dst_a, dst_b, res_v,
                    sem_a, sem_b):
    wid = lax.axis_index("s") * NC + lax.axis_index("c")
    b0 = wid
    b1 = wid + NW

    pltpu.sync_copy(traj_hbm.at[b0], x_v)
    pltpu.sync_copy(traj_hbm.at[B + b0], y_v)
    inb_a = _batch_indices(x_v, y_v, idx_a, b0)
    ga = [pltpu.async_copy(map_hbm.at[idx_a.at[c]], dst_a.at[c], sem_a)
          for c in range(C)]

    pltpu.sync_copy(traj_hbm.at[b1], x_v)
    pltpu.sync_copy(traj_hbm.at[B + b1], y_v)
    inb_b = _batch_indices(x_v, y_v, idx_b, b1)
    gb = [pltpu.async_copy(map_hbm.at[idx_b.at[c]], dst_b.at[c], sem_b)
          for c in range(C)]

    for g in ga:
        g.wait()
    acc = _reduce_batch(dst_a, inb_a)
    res_v[...] = jnp.broadcast_to(jnp.all(acc).astype(jnp.int32), (L,))
    pltpu.sync_copy(res_v, out_hbm.at[b0])

    for g in gb:
        g.wait()
    acc = _reduce_batch(dst_b, inb_b)
    res_v[...] = jnp.broadcast_to(jnp.all(acc).astype(jnp.int32), (L,))
    pltpu.sync_copy(res_v, out_hbm.at[b1])


@jax.jit
def _collision_sc(traj_planes, map_flat):
    kfn = pl.kernel(
        _collision_body,
        out_type=jax.ShapeDtypeStruct((B, L), jnp.int32),
        mesh=plsc.VectorSubcoreMesh(
            core_axis_name="c", subcore_axis_name="s",
            num_cores=NC, num_subcores=NS),
        scratch_types=[
            pltpu.VMEM((T,), jnp.float32),
            pltpu.VMEM((T,), jnp.float32),
            pltpu.VMEM((C, T), jnp.int32),
            pltpu.VMEM((C, T), jnp.int32),
            pltpu.VMEM((C, T), jnp.float32),
            pltpu.VMEM((C, T), jnp.float32),
            pltpu.VMEM((L,), jnp.int32),
            pltpu.SemaphoreType.DMA,
            pltpu.SemaphoreType.DMA,
        ],
        compiler_params=pltpu.CompilerParams(needs_layout_passes=False),
    )
    return kfn(traj_planes, map_flat)


def kernel(trajectory, affordance_map):
    traj_planes = trajectory.transpose(2, 0, 1).reshape(3 * B, T)
    map_flat = (affordance_map
                .reshape(B, H, 2, W // 2, C)
                .transpose(0, 1, 2, 4, 3)
                .reshape(B * H * W * C))
    out = _collision_sc(traj_planes, map_flat)
    return out[:, 0].astype(jnp.bool_)


# speedup vs baseline: 239.2645x; 1.0368x over previous
import jax
import jax.numpy as jnp
from jax import lax
from jax.experimental import pallas as pl
from jax.experimental.pallas import tpu as pltpu
from jax.experimental.pallas import tpu_sc as plsc

B = 64
T = 128
H = 256
W = 256
C = 4
NC = 2
NS = 16
L = 16
NW = NC * NS
NCHUNK = T // L


def _batch_indices(xy_v, idx_v, b):
    in_bounds = []
    for k in range(NCHUNK):
        x = xy_v[0, pl.ds(k * L, L)]
        y = xy_v[1, pl.ds(k * L, L)]
        gx = (((x + 10.0) / 20.0) * float(H)).astype(jnp.int32)
        gy = (((y + 10.0) / 20.0) * float(W)).astype(jnp.int32)
        inb = (gx >= 0) & (gx < H) & (gy >= 0) & (gy < W)
        ic = jnp.clip(gx, 0, H - 1)
        jc = jnp.clip(gy, 0, W - 1)
        e0 = (b * 512 + ic * 2 + (jc >> 7)) * 512 + (jc & 127)
        for c in range(C):
            idx_v[c, pl.ds(k * L, L)] = e0 + c * 128
        in_bounds.append(inb)
    return in_bounds


def _reduce_batch(dst_v, in_bounds):
    acc = None
    for k in range(NCHUNK):
        mass = dst_v[0, pl.ds(k * L, L)]
        for c in range(1, C):
            mass = mass + dst_v[c, pl.ds(k * L, L)]
        ok = in_bounds[k] & jnp.logical_not(mass > 100.0)
        acc = ok if acc is None else (acc & ok)
    return jnp.all(acc).astype(jnp.int32)


def _collision_body(traj_hbm, map_hbm, out_hbm,
                    xy_a, xy_b, idx_a, idx_b, dst_a, dst_b,
                    res_v, stage_v, res8_v, shared, sem_a, sem_b, sem_t):
    cid = lax.axis_index("c")
    sid = lax.axis_index("s")
    b0 = cid * 2 * NS + sid
    b1 = b0 + NS

    ta = pltpu.async_copy(traj_hbm.at[pl.ds(0, 2), b0], xy_a, sem_t)
    tb = pltpu.async_copy(traj_hbm.at[pl.ds(0, 2), b1], xy_b, sem_t)
    ta.wait()
    inb_a = _batch_indices(xy_a, idx_a, b0)
    ga = [pltpu.async_copy(map_hbm.at[idx_a.at[c]], dst_a.at[c], sem_a)
          for c in range(C)]

    tb.wait()
    inb_b = _batch_indices(xy_b, idx_b, b1)
    gb = [pltpu.async_copy(map_hbm.at[idx_b.at[c]], dst_b.at[c], sem_b)
          for c in range(C)]

    for g in ga:
        g.wait()
    v0 = _reduce_batch(dst_a, inb_a)
    for g in gb:
        g.wait()
    v1 = _reduce_batch(dst_b, inb_b)

    lane = lax.iota(jnp.int32, L)
    row = jnp.where(lane == 0, jnp.broadcast_to(v0, (L,)),
                    jnp.where(lane == 1, jnp.broadcast_to(v1, (L,)), 0))
    res_v[...] = row
    pltpu.sync_copy(res_v, shared.at[sid])
    plsc.subcore_barrier()

    @pl.when(sid == 0)
    def _pack():
        pltpu.sync_copy(shared, stage_v)
        kvec = (lane & 3) * 4
        slot = jnp.where(lane < 4, 0, 1)
        packed = jnp.zeros((L,), jnp.int32)
        for c2 in range(4):
            g = plsc.load_gather(stage_v, [kvec + c2, slot])
            packed = packed | (g << (8 * c2))
        packed = jnp.where(lane < 8, packed, 0)
        res8_v[...] = packed
        pltpu.sync_copy(res8_v.at[pl.ds(0, 8)], out_hbm.at[pl.ds(cid * 8, 8)])


@jax.jit
def _collision_sc(traj_planes, map_flat):
    kfn = pl.kernel(
        _collision_body,
        out_type=jax.ShapeDtypeStruct((16,), jnp.int32),
        mesh=plsc.VectorSubcoreMesh(
            core_axis_name="c", subcore_axis_name="s",
            num_cores=NC, num_subcores=NS),
        scratch_types=[
            pltpu.VMEM((2, T), jnp.float32),
            pltpu.VMEM((2, T), jnp.float32),
            pltpu.VMEM((C, T), jnp.int32),
            pltpu.VMEM((C, T), jnp.int32),
            pltpu.VMEM((C, T), jnp.float32),
            pltpu.VMEM((C, T), jnp.float32),
            pltpu.VMEM((L,), jnp.int32),
            pltpu.VMEM((NS, L), jnp.int32),
            pltpu.VMEM((L,), jnp.int32),
            pltpu.VMEM_SHARED((NS, L), jnp.int32),
            pltpu.SemaphoreType.DMA,
            pltpu.SemaphoreType.DMA,
            pltpu.SemaphoreType.DMA,
        ],
        compiler_params=pltpu.CompilerParams(needs_layout_passes=False),
    )
    return kfn(traj_planes, map_flat)


def kernel(trajectory, affordance_map):
    traj_planes = trajectory.transpose(2, 0, 1)
    map_flat = (affordance_map
                .reshape(B, H, 2, W // 2, C)
                .transpose(0, 1, 2, 4, 3)
                .reshape(B * H * W * C))
    out = _collision_sc(traj_planes, map_flat)
    return out.view(jnp.int8).view(jnp.bool_)


# speedup vs baseline: 245.1240x; 1.0245x over previous
import jax
import jax.numpy as jnp
from jax import lax
from jax.experimental import pallas as pl
from jax.experimental.pallas import tpu as pltpu
from jax.experimental.pallas import tpu_sc as plsc

B = 64
T = 128
H = 256
W = 256
C = 4
NC = 2
NS = 16
L = 16
NW = NC * NS
NCHUNK = T // L


def _batch_indices(xy_v, idx_v, b):
    in_bounds = []
    for k in range(NCHUNK):
        x = xy_v[0, pl.ds(k * L, L)]
        y = xy_v[1, pl.ds(k * L, L)]
        gx = (((x + 10.0) / 20.0) * float(H)).astype(jnp.int32)
        gy = (((y + 10.0) / 20.0) * float(W)).astype(jnp.int32)
        inb = (gx >= 0) & (gx < H) & (gy >= 0) & (gy < W)
        ic = jnp.clip(gx, 0, H - 1)
        jc = jnp.clip(gy, 0, W - 1)
        e0 = (b * 512 + ic * 2 + (jc >> 7)) * 512 + (jc & 127)
        for c in range(C):
            idx_v[c, pl.ds(k * L, L)] = e0 + c * 128
        in_bounds.append(inb)
    return in_bounds


def _reduce_batch(dst_v, in_bounds):
    acc = None
    for k in range(NCHUNK):
        mass = dst_v[0, pl.ds(k * L, L)]
        for c in range(1, C):
            mass = mass + dst_v[c, pl.ds(k * L, L)]
        ok = in_bounds[k] & jnp.logical_not(mass > 100.0)
        acc = ok if acc is None else (acc & ok)
    return jnp.all(acc).astype(jnp.int32)


def _collision_body(traj_hbm, map_hbm, out_hbm,
                    xy_a, xy_b, idx_a, idx_b, dst_a, dst_b,
                    res_v, stage_v, res8_v, shared, sem_a, sem_b, sem_t):
    cid = lax.axis_index("c")
    sid = lax.axis_index("s")
    b0 = cid * 2 * NS + sid
    b1 = b0 + NS

    ta = pltpu.async_copy(traj_hbm.at[pl.ds(0, 2), b0], xy_a, sem_t)
    tb = pltpu.async_copy(traj_hbm.at[pl.ds(0, 2), b1], xy_b, sem_t)
    ta.wait()
    inb_a = _batch_indices(xy_a, idx_a, b0)
    ga = [pltpu.async_copy(map_hbm.at[idx_a.at[c]], dst_a.at[c], sem_a)
          for c in range(C)]

    tb.wait()
    inb_b = _batch_indices(xy_b, idx_b, b1)
    gb = [pltpu.async_copy(map_hbm.at[idx_b.at[c]], dst_b.at[c], sem_b)
          for c in range(C)]

    for g in ga:
        g.wait()
    v0 = _reduce_batch(dst_a, inb_a)
    for g in gb:
        g.wait()
    v1 = _reduce_batch(dst_b, inb_b)

    lane = lax.iota(jnp.int32, L)
    row = jnp.where(lane == 0, jnp.broadcast_to(v0, (L,)),
                    jnp.where(lane == 1, jnp.broadcast_to(v1, (L,)), 0))
    res_v[...] = row
    pltpu.sync_copy(res_v, shared.at[sid])
    plsc.subcore_barrier()

    @pl.when(sid == 0)
    def _pack():
        pltpu.sync_copy(shared, stage_v)
        g0 = plsc.load_gather(stage_v, [lane, jnp.zeros((L,), jnp.int32)])
        g1 = plsc.load_gather(stage_v, [lane, jnp.full((L,), 1, jnp.int32)])
        res8_v[pl.ds(0, L)] = g0
        res8_v[pl.ds(L, L)] = g1
        pltpu.sync_copy(res8_v, out_hbm.at[pl.ds(cid * 32, 32)])


@jax.jit
def _collision_sc(traj_planes, map_flat):
    kfn = pl.kernel(
        _collision_body,
        out_type=jax.ShapeDtypeStruct((B,), jnp.int32),
        mesh=plsc.VectorSubcoreMesh(
            core_axis_name="c", subcore_axis_name="s",
            num_cores=NC, num_subcores=NS),
        scratch_types=[
            pltpu.VMEM((2, T), jnp.float32),
            pltpu.VMEM((2, T), jnp.float32),
            pltpu.VMEM((C, T), jnp.int32),
            pltpu.VMEM((C, T), jnp.int32),
            pltpu.VMEM((C, T), jnp.float32),
            pltpu.VMEM((C, T), jnp.float32),
            pltpu.VMEM((L,), jnp.int32),
            pltpu.VMEM((NS, L), jnp.int32),
            pltpu.VMEM((2 * L,), jnp.int32),
            pltpu.VMEM_SHARED((NS, L), jnp.int32),
            pltpu.SemaphoreType.DMA,
            pltpu.SemaphoreType.DMA,
            pltpu.SemaphoreType.DMA,
        ],
        compiler_params=pltpu.CompilerParams(needs_layout_passes=False),
    )
    return kfn(traj_planes, map_flat)


def kernel(trajectory, affordance_map):
    traj_planes = trajectory.transpose(2, 0, 1)
    map_flat = (affordance_map
                .reshape(B, H, 2, W // 2, C)
                .transpose(0, 1, 2, 4, 3)
                .reshape(B * H * W * C))
    out = _collision_sc(traj_planes, map_flat)
    return out.astype(jnp.bool_)
